# baseline (speedup 1.0000x reference)
import jax
import jax.numpy as jnp
from jax.experimental import pallas as pl
from jax.experimental.pallas import tpu as pltpu

MASK_IDX = 999


def _posterior_body(t_sm, gamma_sm, x_ref, xt_ref, o_ref):
    b = pl.program_id(0)
    tb = t_sm[b]
    g = gamma_sm[jnp.maximum(tb - 1, 0)]

    x = x_ref[0]
    mx = jnp.max(x, axis=-1, keepdims=True)
    se = jnp.sum(jnp.exp(x - mx), axis=-1, keepdims=True)
    lse = mx + jnp.log(se)

    m = xt_ref[0, 0] == MASK_IDX
    c = jnp.where(m, 1.0, 1.0 - g)
    off = jnp.log(c) - jnp.log(c + 1e-8)

    o_ref[0] = x - lse + off[:, None]


def kernel(x0_logits, xt, t, gamma):
    B, N, V = x0_logits.shape
    xt3 = xt.reshape(B, 1, N).astype(jnp.int32)

    grid_spec = pltpu.PrefetchScalarGridSpec(
        num_scalar_prefetch=2,
        grid=(B,),
        in_specs=[
            pl.BlockSpec((1, N, V), lambda b, t_sm, g_sm: (b, 0, 0)),
            pl.BlockSpec((1, 1, N), lambda b, t_sm, g_sm: (b, 0, 0)),
        ],
        out_specs=pl.BlockSpec((1, N, V), lambda b, t_sm, g_sm: (b, 0, 0)),
    )

    return pl.pallas_call(
        _posterior_body,
        grid_spec=grid_spec,
        out_shape=jax.ShapeDtypeStruct((B, N, V), jnp.float32),
        compiler_params=pltpu.CompilerParams(
            dimension_semantics=("arbitrary",),
        ),
    )(t.astype(jnp.int32), gamma, x0_logits, xt3)
